# baseline (speedup 1.0000x reference)
import functools

import jax
import jax.numpy as jnp
from jax import lax
from jax.experimental import pallas as pl
from jax.experimental.pallas import tpu as pltpu
from jax.experimental.pallas import tpu_sc as plsc

N = 10000
F_IN = 128
F_H = 100
NC, NS = 2, 16
NW = NC * NS
BIGK = 256
ACC_ROWS = 10240
STRIPE = ACC_ROWS // NS

EPT0 = 10240
EPT1 = 10240
G0 = EPT0 // BIGK
G1 = EPT1 // BIGK
E_PAD = NS * (EPT0 + EPT1)


@functools.cache
def _mesh():
    return plsc.VectorSubcoreMesh(
        core_axis_name="c", subcore_axis_name="s",
        num_cores=NC, num_subcores=NS)


def _tile_base(c, s):
    return jnp.where(c == 0, s * EPT0, NS * EPT0 + s * EPT1)


def _tile_groups(c):
    return jnp.where(c == 0, G0, G1)


@jax.jit
def _sc_degree(dst1d):
    @functools.partial(
        pl.kernel,
        out_type=jax.ShapeDtypeStruct((NC, ACC_ROWS), jnp.float32),
        mesh=_mesh(),
        scratch_types=[
            pltpu.VMEM((BIGK,), jnp.int32),
            pltpu.VMEM((BIGK,), jnp.float32),
            pltpu.VMEM((STRIPE,), jnp.float32),
            pltpu.VMEM_SHARED((ACC_ROWS,), jnp.float32),
        ],
    )
    def deg_kernel(dst_hbm, deg_out, idx_v, ones_v, zbuf, deg_sh):
        c = lax.axis_index("c")
        s = lax.axis_index("s")
        for i in range(BIGK // 16):
            ones_v[pl.ds(i * 16, 16)] = jnp.ones((16,), jnp.float32)
        for i in range(STRIPE // 16):
            zbuf[pl.ds(i * 16, 16)] = jnp.zeros((16,), jnp.float32)
        pltpu.sync_copy(zbuf, deg_sh.at[pl.ds(s * STRIPE, STRIPE)])
        plsc.subcore_barrier()
        base = _tile_base(c, s)

        def body(g, _):
            off = pl.multiple_of(base + g * BIGK, 8)
            pltpu.sync_copy(dst_hbm.at[pl.ds(off, BIGK)], idx_v)
            pltpu.sync_copy(ones_v, deg_sh.at[idx_v], add=True)
            return ()

        lax.fori_loop(0, _tile_groups(c), body, ())
        plsc.subcore_barrier()
        pltpu.sync_copy(deg_sh.at[pl.ds(s * STRIPE, STRIPE)],
                        deg_out.at[c, pl.ds(s * STRIPE, STRIPE)])

    return deg_kernel(dst1d)


@jax.jit
def _sc_aggregate_rows(src1d, dst1d, feat):
    @functools.partial(
        pl.kernel,
        out_type=jax.ShapeDtypeStruct((NC, ACC_ROWS, F_IN), jnp.float32),
        mesh=_mesh(),
        scratch_types=[
            pltpu.VMEM((BIGK,), jnp.int32),
            pltpu.VMEM((BIGK,), jnp.int32),
            pltpu.VMEM((BIGK, F_IN), jnp.float32),
            pltpu.VMEM((16, F_IN), jnp.float32),
            pltpu.VMEM_SHARED((ACC_ROWS, F_IN), jnp.float32),
            pltpu.SemaphoreType.DMA,
        ],
    )
    def agg_kernel(src_hbm, dst_hbm, feat_hbm, out_hbm,
                   src_v, dst_v, rows, zrow, acc_sh, sem):
        c = lax.axis_index("c")
        s = lax.axis_index("s")
        for r in range(16):
            for q in range(F_IN // 16):
                zrow[r, pl.ds(q * 16, 16)] = jnp.zeros((16,), jnp.float32)
        for t in range(STRIPE // 16):
            pltpu.sync_copy(zrow, acc_sh.at[pl.ds(s * STRIPE + t * 16, 16)])
        plsc.subcore_barrier()
        base = _tile_base(c, s)

        def body(g, _):
            off = pl.multiple_of(base + g * BIGK, 8)
            pltpu.sync_copy(src_hbm.at[pl.ds(off, BIGK)], src_v)
            pltpu.sync_copy(dst_hbm.at[pl.ds(off, BIGK)], dst_v)
            pltpu.async_copy(feat_hbm.at[src_v], rows, sem).wait()
            pltpu.sync_copy(rows, acc_sh.at[dst_v], add=True)
            return ()

        lax.fori_loop(0, _tile_groups(c), body, ())
        plsc.subcore_barrier()
        pltpu.sync_copy(acc_sh.at[pl.ds(s * STRIPE, STRIPE)],
                        out_hbm.at[c, pl.ds(s * STRIPE, STRIPE)])

    return agg_kernel(src1d, dst1d, feat)


@jax.jit
def _sc_aggregate_cols2(src1d, dst1d, feat_flat):
    @functools.partial(
        pl.kernel,
        out_type=jax.ShapeDtypeStruct((NC, 2 * ACC_ROWS), jnp.float32),
        mesh=_mesh(),
        scratch_types=[
            pltpu.VMEM((BIGK,), jnp.int32),
            pltpu.VMEM((BIGK,), jnp.int32),
            pltpu.VMEM((BIGK,), jnp.float32),
            pltpu.VMEM((2 * STRIPE,), jnp.float32),
            pltpu.VMEM_SHARED((2 * ACC_ROWS,), jnp.float32),
            pltpu.SemaphoreType.DMA,
        ],
    )
    def agg2_kernel(src_hbm, dst_hbm, feat_hbm, out_hbm,
                    src_v, dst_v, vals, zbuf, acc_sh, sem):
        c = lax.axis_index("c")
        s = lax.axis_index("s")
        for i in range(2 * STRIPE // 16):
            zbuf[pl.ds(i * 16, 16)] = jnp.zeros((16,), jnp.float32)
        pltpu.sync_copy(zbuf, acc_sh.at[pl.ds(s * 2 * STRIPE, 2 * STRIPE)])
        plsc.subcore_barrier()
        base = _tile_base(c, s)
        ngrp = _tile_groups(c)

        def body(t, _):
            g = jnp.where(t < ngrp, t, t - ngrp)
            half = jnp.where(t < ngrp, 0, E_PAD)
            off = pl.multiple_of(half + base + g * BIGK, 8)
            pltpu.sync_copy(src_hbm.at[pl.ds(off, BIGK)], src_v)
            pltpu.sync_copy(dst_hbm.at[pl.ds(off, BIGK)], dst_v)
            pltpu.async_copy(feat_hbm.at[src_v], vals, sem).wait()
            pltpu.sync_copy(vals, acc_sh.at[dst_v], add=True)
            return ()

        lax.fori_loop(0, 2 * ngrp, body, ())
        plsc.subcore_barrier()
        pltpu.sync_copy(acc_sh.at[pl.ds(s * 2 * STRIPE, 2 * STRIPE)],
                        out_hbm.at[c, pl.ds(s * 2 * STRIPE, 2 * STRIPE)])

    return agg2_kernel(src1d, dst1d, feat_flat)


_R = 1000


def _tc1_body(x_ref, ws_ref, bs_ref, degp_ref, xs_ref, d_ref, skip_ref):
    x = x_ref[...]
    deg = degp_ref[0] + degp_ref[1] + 1.0
    d = lax.rsqrt(deg)
    xs_ref[...] = x * d
    d_ref[...] = d
    skip_ref[...] = (
        jnp.dot(x, ws_ref[...], preferred_element_type=jnp.float32)
        + bs_ref[...])


@jax.jit
def _tc1(x, Ws, bs, deg_part):
    grid = N // _R
    return pl.pallas_call(
        _tc1_body,
        grid=(grid,),
        in_specs=[
            pl.BlockSpec((_R, F_IN), lambda i: (i, 0)),
            pl.BlockSpec((F_IN, 2), lambda i: (0, 0)),
            pl.BlockSpec((1, 2), lambda i: (0, 0)),
            pl.BlockSpec((2, _R, 1), lambda i: (0, i, 0)),
        ],
        out_specs=[
            pl.BlockSpec((_R, F_IN), lambda i: (i, 0)),
            pl.BlockSpec((_R, 1), lambda i: (i, 0)),
            pl.BlockSpec((_R, 2), lambda i: (i, 0)),
        ],
        out_shape=[
            jax.ShapeDtypeStruct((N, F_IN), jnp.float32),
            jax.ShapeDtypeStruct((N, 1), jnp.float32),
            jax.ShapeDtypeStruct((N, 2), jnp.float32),
        ],
    )(x, Ws, bs, deg_part)


def _tc2_body(acc_ref, xs_ref, d_ref, w1_ref, b1_ref, w2_ref, hs2_ref):
    d = d_ref[...]
    pre = d * (acc_ref[0] + acc_ref[1] + xs_ref[...])
    h1 = jnp.maximum(
        jnp.dot(pre, w1_ref[...], preferred_element_type=jnp.float32)
        + b1_ref[...], 0.0)
    h2 = jnp.dot(h1, w2_ref[...], preferred_element_type=jnp.float32)
    hs2_ref[...] = h2 * d


@jax.jit
def _tc2(acc1, xs, d, W1, b1, W2):
    grid = N // _R
    return pl.pallas_call(
        _tc2_body,
        grid=(grid,),
        in_specs=[
            pl.BlockSpec((2, _R, F_IN), lambda i: (0, i, 0)),
            pl.BlockSpec((_R, F_IN), lambda i: (i, 0)),
            pl.BlockSpec((_R, 1), lambda i: (i, 0)),
            pl.BlockSpec((F_IN, F_H), lambda i: (0, 0)),
            pl.BlockSpec((1, F_H), lambda i: (0, 0)),
            pl.BlockSpec((F_H, 2), lambda i: (0, 0)),
        ],
        out_specs=pl.BlockSpec((_R, 2), lambda i: (i, 0)),
        out_shape=jax.ShapeDtypeStruct((N, 2), jnp.float32),
    )(acc1, xs, d, W1, b1, W2)


def _tc3_body(acc2_ref, hs2_ref, d_ref, skip_ref, b2_ref, out_ref):
    ssum = acc2_ref[0] + acc2_ref[1] + hs2_ref[...]
    o = d_ref[...] * ssum + b2_ref[...] + skip_ref[...]
    m = jnp.max(o, axis=1, keepdims=True)
    lse = m + jnp.log(jnp.sum(jnp.exp(o - m), axis=1, keepdims=True))
    out_ref[...] = o - lse


@jax.jit
def _tc3(acc2, hs2, d, skip, b2):
    grid = N // _R
    return pl.pallas_call(
        _tc3_body,
        grid=(grid,),
        in_specs=[
            pl.BlockSpec((2, _R, 2), lambda i: (0, i, 0)),
            pl.BlockSpec((_R, 2), lambda i: (i, 0)),
            pl.BlockSpec((_R, 1), lambda i: (i, 0)),
            pl.BlockSpec((_R, 2), lambda i: (i, 0)),
            pl.BlockSpec((1, 2), lambda i: (0, 0)),
        ],
        out_specs=pl.BlockSpec((_R, 2), lambda i: (i, 0)),
        out_shape=jax.ShapeDtypeStruct((N, 2), jnp.float32),
    )(acc2, hs2, d, skip, b2)


def kernel(x, edge_index, W1, b1, W2, b2, Ws, bs):
    e_total = edge_index.shape[1]
    ei = edge_index.astype(jnp.int32)
    pad = E_PAD - e_total
    src = jnp.concatenate([ei[0], jnp.zeros((pad,), jnp.int32)])
    dst = jnp.concatenate([ei[1], jnp.full((pad,), N, jnp.int32)])
    srcC = jnp.concatenate([src, src + N])
    dstC = jnp.concatenate([dst, dst + ACC_ROWS])

    deg_part = _sc_degree(dst)
    degp = deg_part[:, :N].reshape(2, N, 1)
    xs, d, skip = _tc1(x, Ws, bs.reshape(1, 2), degp)
    acc1 = _sc_aggregate_rows(src, dst, xs)
    hs2 = _tc2(acc1[:, :N, :], xs, d, W1, b1.reshape(1, F_H), W2)
    hs2_flat = jnp.transpose(hs2).reshape(2 * N)
    acc2 = _sc_aggregate_cols2(srcC, dstC, hs2_flat)
    acc2t = jnp.transpose(
        acc2.reshape(NC, 2, ACC_ROWS)[:, :, :N], (0, 2, 1))
    return _tc3(acc2t, hs2, d, skip, b2.reshape(1, 2))


# speedup vs baseline: 1.1853x; 1.1853x over previous
import functools

import jax
import jax.numpy as jnp
from jax import lax
from jax.experimental import pallas as pl
from jax.experimental.pallas import tpu as pltpu
from jax.experimental.pallas import tpu_sc as plsc

N = 10000
F_IN = 128
F_H = 100
NC, NS = 2, 16
NW = NC * NS
BIGK = 256
ACC_ROWS = 10240
STRIPE = ACC_ROWS // NS

EPT0 = 10240
EPT1 = 10240
G0 = EPT0 // BIGK
G1 = EPT1 // BIGK
E_PAD = NS * (EPT0 + EPT1)


@functools.cache
def _mesh():
    return plsc.VectorSubcoreMesh(
        core_axis_name="c", subcore_axis_name="s",
        num_cores=NC, num_subcores=NS)


def _tile_base(c, s):
    return jnp.where(c == 0, s * EPT0, NS * EPT0 + s * EPT1)


def _tile_groups(c):
    return jnp.where(c == 0, G0, G1)


@jax.jit
def _sc_degree(dst1d):
    @functools.partial(
        pl.kernel,
        out_type=jax.ShapeDtypeStruct((NC, ACC_ROWS), jnp.float32),
        mesh=_mesh(),
        scratch_types=[
            pltpu.VMEM((EPT0 // 128, 128), jnp.int32),
            pltpu.VMEM((128,), jnp.float32),
            pltpu.VMEM((STRIPE,), jnp.float32),
            pltpu.VMEM_SHARED((ACC_ROWS,), jnp.float32),
        ],
    )
    def deg_kernel(dst_hbm, deg_out, idx_all, ones_v, zbuf, deg_sh):
        c = lax.axis_index("c")
        s = lax.axis_index("s")
        for i in range(128 // 16):
            ones_v[pl.ds(i * 16, 16)] = jnp.ones((16,), jnp.float32)
        for i in range(STRIPE // 16):
            zbuf[pl.ds(i * 16, 16)] = jnp.zeros((16,), jnp.float32)
        wid = c * NS + s
        nrow = EPT0 // 128
        pltpu.sync_copy(dst_hbm.at[pl.ds(wid * nrow, nrow)], idx_all)
        pltpu.sync_copy(zbuf, deg_sh.at[pl.ds(s * STRIPE, STRIPE)])
        plsc.subcore_barrier()

        def body(g, _):
            pltpu.sync_copy(ones_v, deg_sh.at[idx_all.at[g]], add=True)
            return ()

        lax.fori_loop(0, nrow, body, ())
        plsc.subcore_barrier()
        pltpu.sync_copy(deg_sh.at[pl.ds(s * STRIPE, STRIPE)],
                        deg_out.at[c, pl.ds(s * STRIPE, STRIPE)])

    return deg_kernel(dst1d)


@jax.jit
def _sc_aggregate_rows(src1d, dst1d, feat):
    @functools.partial(
        pl.kernel,
        out_type=jax.ShapeDtypeStruct((NC, ACC_ROWS, F_IN), jnp.float32),
        mesh=_mesh(),
        scratch_types=[
            pltpu.VMEM((EPT0,), jnp.int32),
            pltpu.VMEM((EPT0 // 128, 128), jnp.int32),
            pltpu.VMEM((128, F_IN), jnp.float32),
            pltpu.VMEM((16, F_IN), jnp.float32),
            pltpu.VMEM_SHARED((ACC_ROWS, F_IN), jnp.float32),
            pltpu.SemaphoreType.DMA,
        ],
    )
    def agg_kernel(src_hbm, dst_hbm, feat_hbm, out_hbm,
                   src_all, dst_all, rows, zrow, acc_sh, sem):
        c = lax.axis_index("c")
        s = lax.axis_index("s")
        for r in range(16):
            for q in range(F_IN // 16):
                zrow[r, pl.ds(q * 16, 16)] = jnp.zeros((16,), jnp.float32)
        wid = c * NS + s
        nrow = EPT0 // 128
        pltpu.sync_copy(src_hbm.at[pl.ds(wid * EPT0, EPT0)], src_all)
        pltpu.sync_copy(dst_hbm.at[pl.ds(wid * nrow, nrow)], dst_all)
        for t in range(STRIPE // 16):
            pltpu.sync_copy(zrow, acc_sh.at[pl.ds(s * STRIPE + t * 16, 16)])
        plsc.subcore_barrier()

        def body(g, _):
            sl = pl.ds(pl.multiple_of(g * 128, 8), 128)
            pltpu.async_copy(feat_hbm.at[src_all.at[sl]], rows, sem).wait()
            pltpu.sync_copy(rows, acc_sh.at[dst_all.at[g]], add=True)
            return ()

        lax.fori_loop(0, nrow, body, ())
        plsc.subcore_barrier()
        pltpu.sync_copy(acc_sh.at[pl.ds(s * STRIPE, STRIPE)],
                        out_hbm.at[c, pl.ds(s * STRIPE, STRIPE)])

    return agg_kernel(src1d, dst1d, feat)


@jax.jit
def _sc_aggregate_cols2(src1d, dst1d, feat_flat):
    @functools.partial(
        pl.kernel,
        out_type=jax.ShapeDtypeStruct((NC, 2 * ACC_ROWS), jnp.float32),
        mesh=_mesh(),
        scratch_types=[
            pltpu.VMEM((2 * EPT0,), jnp.int32),
            pltpu.VMEM((2 * EPT0 // 128, 128), jnp.int32),
            pltpu.VMEM((BIGK,), jnp.float32),
            pltpu.VMEM((2 * STRIPE,), jnp.float32),
            pltpu.VMEM_SHARED((2 * ACC_ROWS,), jnp.float32),
            pltpu.SemaphoreType.DMA,
        ],
    )
    def agg2_kernel(src_hbm, dst_hbm, feat_hbm, out_hbm,
                    src_all, dst_all, vals, zbuf, acc_sh, sem):
        c = lax.axis_index("c")
        s = lax.axis_index("s")
        for i in range(2 * STRIPE // 16):
            zbuf[pl.ds(i * 16, 16)] = jnp.zeros((16,), jnp.float32)
        wid = c * NS + s
        nrow = EPT0 // 128
        pltpu.sync_copy(src_hbm.at[pl.ds(wid * EPT0, EPT0)],
                        src_all.at[pl.ds(0, EPT0)])
        pltpu.sync_copy(src_hbm.at[pl.ds(E_PAD + wid * EPT0, EPT0)],
                        src_all.at[pl.ds(EPT0, EPT0)])
        pltpu.sync_copy(dst_hbm.at[pl.ds(wid * nrow, nrow)],
                        dst_all.at[pl.ds(0, nrow)])
        pltpu.sync_copy(dst_hbm.at[pl.ds((NW + wid) * nrow, nrow)],
                        dst_all.at[pl.ds(nrow, nrow)])
        pltpu.sync_copy(zbuf, acc_sh.at[pl.ds(s * 2 * STRIPE, 2 * STRIPE)])
        plsc.subcore_barrier()

        def body(t, _):
            sl = pl.ds(pl.multiple_of(t * BIGK, 8), BIGK)
            pltpu.async_copy(feat_hbm.at[src_all.at[sl]], vals, sem).wait()
            pltpu.sync_copy(vals.at[pl.ds(0, 128)],
                            acc_sh.at[dst_all.at[2 * t]], add=True)
            pltpu.sync_copy(vals.at[pl.ds(128, 128)],
                            acc_sh.at[dst_all.at[2 * t + 1]], add=True)
            return ()

        lax.fori_loop(0, 2 * G0, body, ())
        plsc.subcore_barrier()
        pltpu.sync_copy(acc_sh.at[pl.ds(s * 2 * STRIPE, 2 * STRIPE)],
                        out_hbm.at[c, pl.ds(s * 2 * STRIPE, 2 * STRIPE)])

    return agg2_kernel(src1d, dst1d, feat_flat)


_R = 1000


def _tc1_body(x_ref, ws_ref, bs_ref, degp_ref, xs_ref, d_ref, skip_ref):
    x = x_ref[...]
    deg = degp_ref[0] + degp_ref[1] + 1.0
    d = lax.rsqrt(deg)
    xs_ref[...] = x * d
    d_ref[...] = d
    skip_ref[...] = (
        jnp.dot(x, ws_ref[...], preferred_element_type=jnp.float32)
        + bs_ref[...])


@jax.jit
def _tc1(x, Ws, bs, deg_part):
    grid = N // _R
    return pl.pallas_call(
        _tc1_body,
        grid=(grid,),
        in_specs=[
            pl.BlockSpec((_R, F_IN), lambda i: (i, 0)),
            pl.BlockSpec((F_IN, 2), lambda i: (0, 0)),
            pl.BlockSpec((1, 2), lambda i: (0, 0)),
            pl.BlockSpec((2, _R, 1), lambda i: (0, i, 0)),
        ],
        out_specs=[
            pl.BlockSpec((_R, F_IN), lambda i: (i, 0)),
            pl.BlockSpec((_R, 1), lambda i: (i, 0)),
            pl.BlockSpec((_R, 2), lambda i: (i, 0)),
        ],
        out_shape=[
            jax.ShapeDtypeStruct((N, F_IN), jnp.float32),
            jax.ShapeDtypeStruct((N, 1), jnp.float32),
            jax.ShapeDtypeStruct((N, 2), jnp.float32),
        ],
    )(x, Ws, bs, deg_part)


def _tc2_body(acc_ref, xs_ref, d_ref, w1_ref, b1_ref, w2_ref, hs2_ref):
    d = d_ref[...]
    pre = d * (acc_ref[0] + acc_ref[1] + xs_ref[...])
    h1 = jnp.maximum(
        jnp.dot(pre, w1_ref[...], preferred_element_type=jnp.float32)
        + b1_ref[...], 0.0)
    h2 = jnp.dot(h1, w2_ref[...], preferred_element_type=jnp.float32)
    hs2_ref[...] = h2 * d


@jax.jit
def _tc2(acc1, xs, d, W1, b1, W2):
    grid = N // _R
    return pl.pallas_call(
        _tc2_body,
        grid=(grid,),
        in_specs=[
            pl.BlockSpec((2, _R, F_IN), lambda i: (0, i, 0)),
            pl.BlockSpec((_R, F_IN), lambda i: (i, 0)),
            pl.BlockSpec((_R, 1), lambda i: (i, 0)),
            pl.BlockSpec((F_IN, F_H), lambda i: (0, 0)),
            pl.BlockSpec((1, F_H), lambda i: (0, 0)),
            pl.BlockSpec((F_H, 2), lambda i: (0, 0)),
        ],
        out_specs=pl.BlockSpec((_R, 2), lambda i: (i, 0)),
        out_shape=jax.ShapeDtypeStruct((N, 2), jnp.float32),
    )(acc1, xs, d, W1, b1, W2)


def _tc3_body(acc2_ref, hs2_ref, d_ref, skip_ref, b2_ref, out_ref):
    ssum = acc2_ref[0] + acc2_ref[1] + hs2_ref[...]
    o = d_ref[...] * ssum + b2_ref[...] + skip_ref[...]
    m = jnp.max(o, axis=1, keepdims=True)
    lse = m + jnp.log(jnp.sum(jnp.exp(o - m), axis=1, keepdims=True))
    out_ref[...] = o - lse


@jax.jit
def _tc3(acc2, hs2, d, skip, b2):
    grid = N // _R
    return pl.pallas_call(
        _tc3_body,
        grid=(grid,),
        in_specs=[
            pl.BlockSpec((2, _R, 2), lambda i: (0, i, 0)),
            pl.BlockSpec((_R, 2), lambda i: (i, 0)),
            pl.BlockSpec((_R, 1), lambda i: (i, 0)),
            pl.BlockSpec((_R, 2), lambda i: (i, 0)),
            pl.BlockSpec((1, 2), lambda i: (0, 0)),
        ],
        out_specs=pl.BlockSpec((_R, 2), lambda i: (i, 0)),
        out_shape=jax.ShapeDtypeStruct((N, 2), jnp.float32),
    )(acc2, hs2, d, skip, b2)


def kernel(x, edge_index, W1, b1, W2, b2, Ws, bs):
    e_total = edge_index.shape[1]
    ei = edge_index.astype(jnp.int32)
    pad = E_PAD - e_total
    src = jnp.concatenate([ei[0], jnp.zeros((pad,), jnp.int32)])
    dst = jnp.concatenate([ei[1], jnp.full((pad,), N, jnp.int32)])
    srcC = jnp.concatenate([src, src + N])
    dstC = jnp.concatenate([dst, dst + ACC_ROWS])

    src2 = src
    dst2 = dst.reshape(NW * (EPT0 // 128), 128)
    srcC2 = srcC
    dstC2 = dstC.reshape(2 * NW * (EPT0 // 128), 128)

    deg_part = _sc_degree(dst2)
    degp = deg_part[:, :N].reshape(2, N, 1)
    xs, d, skip = _tc1(x, Ws, bs.reshape(1, 2), degp)
    acc1 = _sc_aggregate_rows(src2, dst2, xs)
    hs2 = _tc2(acc1[:, :N, :], xs, d, W1, b1.reshape(1, F_H), W2)
    hs2_flat = jnp.transpose(hs2).reshape(2 * N)
    acc2 = _sc_aggregate_cols2(srcC2, dstC2, hs2_flat)
    acc2t = jnp.transpose(
        acc2.reshape(NC, 2, ACC_ROWS)[:, :, :N], (0, 2, 1))
    return _tc3(acc2t, hs2, d, skip, b2.reshape(1, 2))


# speedup vs baseline: 1.3319x; 1.1238x over previous
import functools

import jax
import jax.numpy as jnp
from jax import lax
from jax.experimental import pallas as pl
from jax.experimental.pallas import tpu as pltpu
from jax.experimental.pallas import tpu_sc as plsc

N = 10000
F_IN = 128
F_H = 100
NC, NS = 2, 16
NW = NC * NS
K = 128
NB = 4
NB2 = 8
ACC_ROWS = 10240
STRIPE = ACC_ROWS // NS


@functools.cache
def _mesh():
    return plsc.VectorSubcoreMesh(
        core_axis_name="c", subcore_axis_name="s",
        num_cores=NC, num_subcores=NS)


def _edges_per_tile(e_total):
    per = -(-e_total // NW)
    per = -(-per // (K * NB)) * (K * NB)
    return per


@functools.partial(jax.jit, static_argnames=("ept",))
def _sc_degree(dst2d, *, ept):
    chunks = ept // K

    @functools.partial(
        pl.kernel,
        out_type=jax.ShapeDtypeStruct((NC, ACC_ROWS), jnp.float32),
        mesh=_mesh(),
        scratch_types=[
            pltpu.VMEM((chunks, K), jnp.int32),
            pltpu.VMEM((K,), jnp.float32),
            pltpu.VMEM((STRIPE,), jnp.float32),
            pltpu.VMEM_SHARED((ACC_ROWS,), jnp.float32),
        ],
    )
    def deg_kernel(dst_hbm, deg_out, idx_all, ones_v, zbuf, deg_sh):
        c = lax.axis_index("c")
        s = lax.axis_index("s")
        wid = c * NS + s
        for i in range(K // 16):
            ones_v[pl.ds(i * 16, 16)] = jnp.ones((16,), jnp.float32)
        for i in range(STRIPE // 16):
            zbuf[pl.ds(i * 16, 16)] = jnp.zeros((16,), jnp.float32)
        pltpu.sync_copy(dst_hbm.at[pl.ds(wid * chunks, chunks)], idx_all)
        pltpu.sync_copy(zbuf, deg_sh.at[pl.ds(s * STRIPE, STRIPE)])
        plsc.subcore_barrier()

        def body(j, _):
            pltpu.sync_copy(ones_v, deg_sh.at[idx_all.at[j]], add=True)
            return ()

        lax.fori_loop(0, chunks, body, ())
        plsc.subcore_barrier()
        pltpu.sync_copy(deg_sh.at[pl.ds(s * STRIPE, STRIPE)],
                        deg_out.at[c, pl.ds(s * STRIPE, STRIPE)])

    return deg_kernel(dst2d)


BIGK = K * 2


@functools.partial(jax.jit, static_argnames=("ept",))
def _sc_aggregate_rows(src1d, dst1d, feat, *, ept):
    groups = ept // BIGK

    @functools.partial(
        pl.kernel,
        out_type=jax.ShapeDtypeStruct((NC, ACC_ROWS, F_IN), jnp.float32),
        mesh=_mesh(),
        scratch_types=[
            pltpu.VMEM((ept,), jnp.int32),
            pltpu.VMEM((BIGK,), jnp.int32),
            pltpu.VMEM((BIGK, F_IN), jnp.float32),
            pltpu.VMEM((16, F_IN), jnp.float32),
            pltpu.VMEM_SHARED((ACC_ROWS, F_IN), jnp.float32),
            pltpu.SemaphoreType.DMA,
            pltpu.SemaphoreType.DMA,
        ],
    )
    def agg_kernel(src_hbm, dst_hbm, feat_hbm, out_hbm,
                   src_full, dst_grp, rows, zrow, acc_sh, sem, sem2):
        c = lax.axis_index("c")
        s = lax.axis_index("s")
        wid = c * NS + s
        for r in range(16):
            for q in range(F_IN // 16):
                zrow[r, pl.ds(q * 16, 16)] = jnp.zeros((16,), jnp.float32)
        pltpu.sync_copy(src_hbm.at[pl.ds(wid * ept, ept)], src_full)
        for t in range(STRIPE // 16):
            pltpu.sync_copy(zrow, acc_sh.at[pl.ds(s * STRIPE + t * 16, 16)])
        plsc.subcore_barrier()

        def group(g, _):
            sl = pl.ds(pl.multiple_of(g * BIGK, 8), BIGK)
            dcp = pltpu.async_copy(
                dst_hbm.at[pl.ds(pl.multiple_of(wid * ept + g * BIGK, 8),
                                 BIGK)], dst_grp, sem2)
            gcp = pltpu.async_copy(feat_hbm.at[src_full.at[sl]], rows, sem)
            dcp.wait()
            gcp.wait()
            pltpu.sync_copy(rows, acc_sh.at[dst_grp], add=True)
            return ()

        lax.fori_loop(0, groups, group, ())
        plsc.subcore_barrier()
        pltpu.sync_copy(acc_sh.at[pl.ds(s * STRIPE, STRIPE)],
                        out_hbm.at[c, pl.ds(s * STRIPE, STRIPE)])

    return agg_kernel(src1d, dst1d, feat)


@functools.partial(jax.jit, static_argnames=("ept",))
def _sc_aggregate_cols2(srcC, dstC, feat_flat, *, ept):
    chunks = ept // K
    total = 2 * chunks
    groups = total // NB2

    @functools.partial(
        pl.kernel,
        out_type=jax.ShapeDtypeStruct((NC, 2 * ACC_ROWS), jnp.float32),
        mesh=_mesh(),
        scratch_types=[
            pltpu.VMEM((total, K), jnp.int32),
            pltpu.VMEM((total, K), jnp.int32),
            [pltpu.VMEM((K,), jnp.float32) for _ in range(NB2)],
            pltpu.VMEM((2 * STRIPE,), jnp.float32),
            pltpu.VMEM_SHARED((2 * ACC_ROWS,), jnp.float32),
            [pltpu.SemaphoreType.DMA for _ in range(NB2)],
        ],
    )
    def agg2_kernel(src_hbm, dst_hbm, feat_hbm, out_hbm,
                    src_all, dst_all, vals, zbuf, acc_sh, sems):
        c = lax.axis_index("c")
        s = lax.axis_index("s")
        wid = c * NS + s
        for i in range(2 * STRIPE // 16):
            zbuf[pl.ds(i * 16, 16)] = jnp.zeros((16,), jnp.float32)
        half = NW * chunks
        pltpu.sync_copy(src_hbm.at[pl.ds(wid * chunks, chunks)],
                        src_all.at[pl.ds(0, chunks)])
        pltpu.sync_copy(src_hbm.at[pl.ds(half + wid * chunks, chunks)],
                        src_all.at[pl.ds(chunks, chunks)])
        pltpu.sync_copy(dst_hbm.at[pl.ds(wid * chunks, chunks)],
                        dst_all.at[pl.ds(0, chunks)])
        pltpu.sync_copy(dst_hbm.at[pl.ds(half + wid * chunks, chunks)],
                        dst_all.at[pl.ds(chunks, chunks)])
        pltpu.sync_copy(zbuf, acc_sh.at[pl.ds(s * 2 * STRIPE, 2 * STRIPE)])
        plsc.subcore_barrier()
        for b in range(NB2):
            pltpu.async_copy(feat_hbm.at[src_all.at[b]], vals[b], sems[b])

        def group(g, _):
            for b in range(NB2):
                j = g * NB2 + b
                pltpu.make_async_copy(
                    feat_hbm.at[src_all.at[j]], vals[b], sems[b]).wait()
                pltpu.sync_copy(vals[b], acc_sh.at[dst_all.at[j]], add=True)
                nxt = j + NB2

                @pl.when(nxt < total)
                def _():
                    pltpu.async_copy(
                        feat_hbm.at[src_all.at[nxt]], vals[b], sems[b])
            return ()

        lax.fori_loop(0, groups, group, ())
        plsc.subcore_barrier()
        pltpu.sync_copy(acc_sh.at[pl.ds(s * 2 * STRIPE, 2 * STRIPE)],
                        out_hbm.at[c, pl.ds(s * 2 * STRIPE, 2 * STRIPE)])

    return agg2_kernel(srcC, dstC, feat_flat)


_R = 1000


def _tc1_body(x_ref, ws_ref, bs_ref, degp_ref, xs_ref, d_ref, skip_ref):
    x = x_ref[...]
    deg = degp_ref[0] + degp_ref[1] + 1.0
    d = lax.rsqrt(deg)
    xs_ref[...] = x * d
    d_ref[...] = d
    skip_ref[...] = (
        jnp.dot(x, ws_ref[...], preferred_element_type=jnp.float32)
        + bs_ref[...])


@jax.jit
def _tc1(x, Ws, bs, deg_part):
    grid = N // _R
    return pl.pallas_call(
        _tc1_body,
        grid=(grid,),
        in_specs=[
            pl.BlockSpec((_R, F_IN), lambda i: (i, 0)),
            pl.BlockSpec((F_IN, 2), lambda i: (0, 0)),
            pl.BlockSpec((1, 2), lambda i: (0, 0)),
            pl.BlockSpec((2, _R, 1), lambda i: (0, i, 0)),
        ],
        out_specs=[
            pl.BlockSpec((_R, F_IN), lambda i: (i, 0)),
            pl.BlockSpec((_R, 1), lambda i: (i, 0)),
            pl.BlockSpec((_R, 2), lambda i: (i, 0)),
        ],
        out_shape=[
            jax.ShapeDtypeStruct((N, F_IN), jnp.float32),
            jax.ShapeDtypeStruct((N, 1), jnp.float32),
            jax.ShapeDtypeStruct((N, 2), jnp.float32),
        ],
    )(x, Ws, bs, deg_part)


def _tc2_body(acc_ref, xs_ref, d_ref, w1_ref, b1_ref, w2_ref, hs2_ref):
    d = d_ref[...]
    pre = d * (acc_ref[0] + acc_ref[1] + xs_ref[...])
    h1 = jnp.maximum(
        jnp.dot(pre, w1_ref[...], preferred_element_type=jnp.float32)
        + b1_ref[...], 0.0)
    h2 = jnp.dot(h1, w2_ref[...], preferred_element_type=jnp.float32)
    hs2_ref[...] = h2 * d


@jax.jit
def _tc2(acc1, xs, d, W1, b1, W2):
    grid = N // _R
    return pl.pallas_call(
        _tc2_body,
        grid=(grid,),
        in_specs=[
            pl.BlockSpec((2, _R, F_IN), lambda i: (0, i, 0)),
            pl.BlockSpec((_R, F_IN), lambda i: (i, 0)),
            pl.BlockSpec((_R, 1), lambda i: (i, 0)),
            pl.BlockSpec((F_IN, F_H), lambda i: (0, 0)),
            pl.BlockSpec((1, F_H), lambda i: (0, 0)),
            pl.BlockSpec((F_H, 2), lambda i: (0, 0)),
        ],
        out_specs=pl.BlockSpec((_R, 2), lambda i: (i, 0)),
        out_shape=jax.ShapeDtypeStruct((N, 2), jnp.float32),
    )(acc1, xs, d, W1, b1, W2)


def _tc3_body(acc2_ref, hs2_ref, d_ref, skip_ref, b2_ref, out_ref):
    ssum = acc2_ref[0] + acc2_ref[1] + hs2_ref[...]
    o = d_ref[...] * ssum + b2_ref[...] + skip_ref[...]
    m = jnp.max(o, axis=1, keepdims=True)
    lse = m + jnp.log(jnp.sum(jnp.exp(o - m), axis=1, keepdims=True))
    out_ref[...] = o - lse


@jax.jit
def _tc3(acc2, hs2, d, skip, b2):
    grid = N // _R
    return pl.pallas_call(
        _tc3_body,
        grid=(grid,),
        in_specs=[
            pl.BlockSpec((2, _R, 2), lambda i: (0, i, 0)),
            pl.BlockSpec((_R, 2), lambda i: (i, 0)),
            pl.BlockSpec((_R, 1), lambda i: (i, 0)),
            pl.BlockSpec((_R, 2), lambda i: (i, 0)),
            pl.BlockSpec((1, 2), lambda i: (0, 0)),
        ],
        out_specs=pl.BlockSpec((_R, 2), lambda i: (i, 0)),
        out_shape=jax.ShapeDtypeStruct((N, 2), jnp.float32),
    )(acc2, hs2, d, skip, b2)


def kernel(x, edge_index, W1, b1, W2, b2, Ws, bs):
    e_total = edge_index.shape[1]
    ept = _edges_per_tile(e_total)
    e_pad = ept * NW
    ei = edge_index.astype(jnp.int32)
    pad = e_pad - e_total
    src = jnp.concatenate([ei[0], jnp.zeros((pad,), jnp.int32)])
    dst = jnp.concatenate([ei[1], jnp.full((pad,), N, jnp.int32)])

    chunks = ept // K
    src2d = src.reshape(NW * chunks, K)
    dst2d = dst.reshape(NW * chunks, K)
    srcC = jnp.concatenate([src2d, src2d + N])
    dstC = jnp.concatenate([dst2d, dst2d + ACC_ROWS])

    deg_part = _sc_degree(dst2d, ept=ept)
    degp = deg_part[:, :N].reshape(2, N, 1)
    xs, d, skip = _tc1(x, Ws, bs.reshape(1, 2), degp)
    acc1 = _sc_aggregate_rows(src, dst, xs, ept=ept)
    hs2 = _tc2(acc1[:, :N, :], xs, d, W1, b1.reshape(1, F_H), W2)
    hs2_flat = jnp.transpose(hs2).reshape(2 * N)
    acc2 = _sc_aggregate_cols2(srcC, dstC, hs2_flat, ept=ept)
    acc2t = jnp.transpose(
        acc2.reshape(NC, 2, ACC_ROWS)[:, :, :N], (0, 2, 1))
    return _tc3(acc2t, hs2, d, skip, b2.reshape(1, 2))
